# baseline (speedup 1.0000x reference)
import jax
import jax.numpy as jnp
from jax.experimental import pallas as pl

TOPK = 8
NUM_EXPERTS = 64


def _gating_kernel(x_ref, w_ref, gates_ref, load_ref):
    step = pl.program_id(0)
    logits = jnp.dot(x_ref[...], w_ref[...], preferred_element_type=jnp.float32)
    m = jnp.max(logits, axis=1, keepdims=True)
    e = jnp.exp(logits - m)
    p = e / jnp.sum(e, axis=1, keepdims=True)

    neg_inf = jnp.float32(-jnp.inf)
    vals = p
    pmax = None
    for i in range(TOPK):
        vmax = jnp.max(vals, axis=1, keepdims=True)
        if i == 0:
            pmax = vmax
        vals = jnp.where(vals == vmax, neg_inf, vals)

    sel = vals < 0.0
    e2 = jnp.where(sel, jnp.exp(p - pmax), 0.0)
    gates = e2 / jnp.sum(e2, axis=1, keepdims=True)
    gates_ref[...] = gates

    cnt = jnp.sum(sel.astype(jnp.int32), axis=0, keepdims=True)

    @pl.when(step == 0)
    def _init():
        load_ref[...] = cnt

    @pl.when(step != 0)
    def _acc():
        load_ref[...] += cnt


def kernel(x, w_gate, train):
    del train
    tokens, d = x.shape
    block = 1024
    grid = tokens // block
    gates, load = pl.pallas_call(
        _gating_kernel,
        grid=(grid,),
        in_specs=[
            pl.BlockSpec((block, d), lambda i: (i, 0)),
            pl.BlockSpec((d, NUM_EXPERTS), lambda i: (0, 0)),
        ],
        out_specs=[
            pl.BlockSpec((block, NUM_EXPERTS), lambda i: (i, 0)),
            pl.BlockSpec((1, NUM_EXPERTS), lambda i: (0, 0)),
        ],
        out_shape=[
            jax.ShapeDtypeStruct((tokens, NUM_EXPERTS), jnp.float32),
            jax.ShapeDtypeStruct((1, NUM_EXPERTS), jnp.int32),
        ],
    )(x, w_gate)
    return gates, load.reshape(NUM_EXPERTS)


# speedup vs baseline: 1.5081x; 1.5081x over previous
import jax
import jax.numpy as jnp
from jax.experimental import pallas as pl

TOPK = 8
NUM_EXPERTS = 64


def _gating_kernel(x_ref, w_ref, gates_ref, load_ref):
    step = pl.program_id(0)
    logits = jax.lax.dot_general(
        w_ref[...], x_ref[...],
        dimension_numbers=(((0,), (1,)), ((), ())),
        preferred_element_type=jnp.float32)
    m = jnp.max(logits, axis=0, keepdims=True)
    e = jnp.exp(logits - m)
    p = e / jnp.sum(e, axis=0, keepdims=True)

    neg_inf = jnp.float32(-jnp.inf)
    vals = p
    pmax = None
    for i in range(TOPK):
        vmax = jnp.max(vals, axis=0, keepdims=True)
        if i == 0:
            pmax = vmax
        vals = jnp.where(vals == vmax, neg_inf, vals)

    sel = vals < 0.0
    e2 = jnp.where(sel, jnp.exp(p - pmax), 0.0)
    gates_t = e2 / jnp.sum(e2, axis=0, keepdims=True)
    gates_ref[...] = gates_t.T

    cnt = jnp.sum(sel.astype(jnp.int32), axis=1, keepdims=True)

    @pl.when(step == 0)
    def _init():
        load_ref[...] = cnt

    @pl.when(step != 0)
    def _acc():
        load_ref[...] += cnt


def kernel(x, w_gate, train):
    del train
    tokens, d = x.shape
    block = 4096
    grid = tokens // block
    gates, load = pl.pallas_call(
        _gating_kernel,
        grid=(grid,),
        in_specs=[
            pl.BlockSpec((block, d), lambda i: (i, 0)),
            pl.BlockSpec((d, NUM_EXPERTS), lambda i: (0, 0)),
        ],
        out_specs=[
            pl.BlockSpec((block, NUM_EXPERTS), lambda i: (i, 0)),
            pl.BlockSpec((NUM_EXPERTS, 1), lambda i: (0, 0)),
        ],
        out_shape=[
            jax.ShapeDtypeStruct((tokens, NUM_EXPERTS), jnp.float32),
            jax.ShapeDtypeStruct((NUM_EXPERTS, 1), jnp.int32),
        ],
    )(x, w_gate)
    return gates, load.reshape(NUM_EXPERTS)


# speedup vs baseline: 1.5154x; 1.0049x over previous
import jax
import jax.numpy as jnp
from jax.experimental import pallas as pl
from jax.experimental.pallas import tpu as pltpu

TOPK = 8
NUM_EXPERTS = 64


def _gating_kernel(x_ref, w_ref, gates_ref, load_ref):
    logits = jax.lax.dot_general(
        w_ref[...], x_ref[...],
        dimension_numbers=(((0,), (1,)), ((), ())),
        preferred_element_type=jnp.float32)
    m = jnp.max(logits, axis=0, keepdims=True)
    e = jnp.exp(logits - m)
    p = e / jnp.sum(e, axis=0, keepdims=True)

    neg_inf = jnp.float32(-jnp.inf)
    vals = p
    pmax = None
    for i in range(TOPK):
        vmax = jnp.max(vals, axis=0, keepdims=True)
        if i == 0:
            pmax = vmax
        vals = jnp.where(vals == vmax, neg_inf, vals)

    sel = vals < 0.0
    e2 = jnp.where(sel, jnp.exp(p - pmax), 0.0)
    gates_t = e2 / jnp.sum(e2, axis=0, keepdims=True)
    gates_ref[...] = gates_t.T

    load_ref[0, ...] = jnp.sum(sel.astype(jnp.int32), axis=1, keepdims=True)


def kernel(x, w_gate, train):
    del train
    tokens, d = x.shape
    block = 4096
    grid = tokens // block
    gates, load_parts = pl.pallas_call(
        _gating_kernel,
        grid=(grid,),
        in_specs=[
            pl.BlockSpec((block, d), lambda i: (i, 0)),
            pl.BlockSpec((d, NUM_EXPERTS), lambda i: (0, 0)),
        ],
        out_specs=[
            pl.BlockSpec((block, NUM_EXPERTS), lambda i: (i, 0)),
            pl.BlockSpec((1, NUM_EXPERTS, 1), lambda i: (i, 0, 0)),
        ],
        out_shape=[
            jax.ShapeDtypeStruct((tokens, NUM_EXPERTS), jnp.float32),
            jax.ShapeDtypeStruct((grid, NUM_EXPERTS, 1), jnp.int32),
        ],
        compiler_params=pltpu.CompilerParams(
            dimension_semantics=("parallel",)),
    )(x, w_gate)
    return gates, load_parts.sum(axis=0).reshape(NUM_EXPERTS)
